# Initial kernel scaffold; baseline (speedup 1.0000x reference)
import jax
import jax.numpy as jnp
from jax.experimental import pallas as pl


def kernel(x, W1, b1, Wout, bout, coefs, intercepts):
    raise NotImplementedError("write your pallas kernel here")


# speedup vs baseline: 1.9383x; 1.9383x over previous
import functools

import jax
import jax.numpy as jnp
from jax.experimental import pallas as pl
from jax.experimental.pallas import tpu as pltpu

_B, _D, _E = 8192, 1024, 16
_TOPK = 2
_LOSS_COEF = 0.01


def _fused_kernel(x_ref, w1t_ref, b1_ref, woutt_ref, bout_ref, coefst_ref,
                  icpt_ref, out_ref, loss_ref, gates_ref, imp_ref, load_ref,
                  *, n_steps):
    i = pl.program_id(0)

    x = x_ref[...]
    g = jnp.dot(x, w1t_ref[...], preferred_element_type=jnp.float32)
    g = jnp.maximum(g + b1_ref[...], 0.0)
    scores = jnp.dot(g, woutt_ref[...], preferred_element_type=jnp.float32)
    scores = scores + bout_ref[...]

    e_idx = jax.lax.broadcasted_iota(jnp.int32, scores.shape, 1)
    m1 = jnp.max(scores, axis=1, keepdims=True)
    idx1 = jnp.min(jnp.where(scores == m1, e_idx, _E), axis=1, keepdims=True)
    sel1 = e_idx == idx1
    rest = jnp.where(sel1, -jnp.inf, scores)
    m2 = jnp.max(rest, axis=1, keepdims=True)
    idx2 = jnp.min(jnp.where(rest == m2, e_idx, _E), axis=1, keepdims=True)
    mask = sel1 | (e_idx == idx2)

    masked = jnp.where(mask, scores, 0.0)
    mx = jnp.max(masked, axis=1, keepdims=True)
    ex = jnp.exp(masked - mx)
    gates = ex / jnp.sum(ex, axis=1, keepdims=True)
    gates_ref[...] = gates

    eo = jnp.dot(x, coefst_ref[...], preferred_element_type=jnp.float32)
    eo = eo + icpt_ref[...]
    out_ref[...] = jnp.sum(gates * eo, axis=1, keepdims=True)

    @pl.when(i == 0)
    def _init():
        imp_ref[...] = jnp.zeros_like(imp_ref)
        load_ref[...] = jnp.zeros_like(load_ref)

    imp_ref[...] += jnp.sum(gates, axis=0, keepdims=True)
    load_ref[...] += jnp.sum((gates > 0.0).astype(jnp.float32), axis=0,
                             keepdims=True)

    @pl.when(i == n_steps - 1)
    def _finish():
        def cv2(v):
            mean = jnp.sum(v) / _E
            var = jnp.sum((v - mean) ** 2) / (_E - 1)
            return var / (mean * mean + 1e-10)

        loss = (cv2(imp_ref[...]) + cv2(load_ref[...])) * _LOSS_COEF
        loss_ref[...] = loss.reshape(1, 1)


@jax.jit
def kernel(x, W1, b1, Wout, bout, coefs, intercepts):
    BT = 512
    n_steps = _B // BT

    w1t = W1.T
    woutt = Wout.T
    coefst = coefs.T
    b1r = b1.reshape(1, _D)
    boutr = bout.reshape(1, _E)
    icptr = intercepts.reshape(1, _E)

    out2d, loss2d, gates = pl.pallas_call(
        functools.partial(_fused_kernel, n_steps=n_steps),
        grid=(n_steps,),
        in_specs=[
            pl.BlockSpec((BT, _D), lambda i: (i, 0)),
            pl.BlockSpec((_D, _D), lambda i: (0, 0)),
            pl.BlockSpec((1, _D), lambda i: (0, 0)),
            pl.BlockSpec((_D, _E), lambda i: (0, 0)),
            pl.BlockSpec((1, _E), lambda i: (0, 0)),
            pl.BlockSpec((_D, _E), lambda i: (0, 0)),
            pl.BlockSpec((1, _E), lambda i: (0, 0)),
        ],
        out_specs=[
            pl.BlockSpec((BT, 1), lambda i: (i, 0)),
            pl.BlockSpec((1, 1), lambda i: (0, 0)),
            pl.BlockSpec((BT, _E), lambda i: (i, 0)),
        ],
        out_shape=[
            jax.ShapeDtypeStruct((_B, 1), jnp.float32),
            jax.ShapeDtypeStruct((1, 1), jnp.float32),
            jax.ShapeDtypeStruct((_B, _E), jnp.float32),
        ],
        scratch_shapes=[
            pltpu.VMEM((1, _E), jnp.float32),
            pltpu.VMEM((1, _E), jnp.float32),
        ],
        compiler_params=pltpu.CompilerParams(
            dimension_semantics=("arbitrary",),
        ),
    )(x, w1t, b1r, woutt, boutr, coefst, icptr)

    return out2d.reshape(_B), loss2d[0, 0], gates


# speedup vs baseline: 1.9556x; 1.0089x over previous
import functools

import jax
import jax.numpy as jnp
from jax.experimental import pallas as pl
from jax.experimental.pallas import tpu as pltpu

_B, _D, _E = 8192, 1024, 16
_TOPK = 2
_LOSS_COEF = 0.01


def _fused_kernel(x_ref, w1t_ref, b1_ref, woutt_ref, bout_ref, coefst_ref,
                  icpt_ref, out_ref, loss_ref, gates_ref, imp_ref, load_ref,
                  *, n_steps):
    i = pl.program_id(0)

    x = x_ref[...]
    g = jnp.dot(x.astype(jnp.bfloat16), w1t_ref[...].astype(jnp.bfloat16),
                preferred_element_type=jnp.float32)
    g = jnp.maximum(g + b1_ref[...], 0.0)
    scores = jnp.dot(g, woutt_ref[...], preferred_element_type=jnp.float32)
    scores = scores + bout_ref[...]

    e_idx = jax.lax.broadcasted_iota(jnp.int32, scores.shape, 1)
    m1 = jnp.max(scores, axis=1, keepdims=True)
    idx1 = jnp.min(jnp.where(scores == m1, e_idx, _E), axis=1, keepdims=True)
    sel1 = e_idx == idx1
    rest = jnp.where(sel1, -jnp.inf, scores)
    m2 = jnp.max(rest, axis=1, keepdims=True)
    idx2 = jnp.min(jnp.where(rest == m2, e_idx, _E), axis=1, keepdims=True)
    mask = sel1 | (e_idx == idx2)

    masked = jnp.where(mask, scores, 0.0)
    mx = jnp.max(masked, axis=1, keepdims=True)
    ex = jnp.exp(masked - mx)
    gates = ex / jnp.sum(ex, axis=1, keepdims=True)
    gates_ref[...] = gates

    eo = jnp.dot(x, coefst_ref[...], preferred_element_type=jnp.float32)
    eo = eo + icpt_ref[...]
    out_ref[...] = jnp.sum(gates * eo, axis=1, keepdims=True)

    @pl.when(i == 0)
    def _init():
        imp_ref[...] = jnp.zeros_like(imp_ref)
        load_ref[...] = jnp.zeros_like(load_ref)

    imp_ref[...] += jnp.sum(gates, axis=0, keepdims=True)
    load_ref[...] += jnp.sum((gates > 0.0).astype(jnp.float32), axis=0,
                             keepdims=True)

    @pl.when(i == n_steps - 1)
    def _finish():
        def cv2(v):
            mean = jnp.sum(v) / _E
            var = jnp.sum((v - mean) ** 2) / (_E - 1)
            return var / (mean * mean + 1e-10)

        loss = (cv2(imp_ref[...]) + cv2(load_ref[...])) * _LOSS_COEF
        loss_ref[...] = loss.reshape(1, 1)


@jax.jit
def kernel(x, W1, b1, Wout, bout, coefs, intercepts):
    BT = 512
    n_steps = _B // BT

    w1t = W1.T
    woutt = Wout.T
    coefst = coefs.T
    b1r = b1.reshape(1, _D)
    boutr = bout.reshape(1, _E)
    icptr = intercepts.reshape(1, _E)

    out2d, loss2d, gates = pl.pallas_call(
        functools.partial(_fused_kernel, n_steps=n_steps),
        grid=(n_steps,),
        in_specs=[
            pl.BlockSpec((BT, _D), lambda i: (i, 0)),
            pl.BlockSpec((_D, _D), lambda i: (0, 0)),
            pl.BlockSpec((1, _D), lambda i: (0, 0)),
            pl.BlockSpec((_D, _E), lambda i: (0, 0)),
            pl.BlockSpec((1, _E), lambda i: (0, 0)),
            pl.BlockSpec((_D, _E), lambda i: (0, 0)),
            pl.BlockSpec((1, _E), lambda i: (0, 0)),
        ],
        out_specs=[
            pl.BlockSpec((BT, 1), lambda i: (i, 0)),
            pl.BlockSpec((1, 1), lambda i: (0, 0)),
            pl.BlockSpec((BT, _E), lambda i: (i, 0)),
        ],
        out_shape=[
            jax.ShapeDtypeStruct((_B, 1), jnp.float32),
            jax.ShapeDtypeStruct((1, 1), jnp.float32),
            jax.ShapeDtypeStruct((_B, _E), jnp.float32),
        ],
        scratch_shapes=[
            pltpu.VMEM((1, _E), jnp.float32),
            pltpu.VMEM((1, _E), jnp.float32),
        ],
        compiler_params=pltpu.CompilerParams(
            dimension_semantics=("arbitrary",),
        ),
    )(x, w1t, b1r, woutt, boutr, coefst, icptr)

    return out2d.reshape(_B), loss2d[0, 0], gates


# speedup vs baseline: 2.7135x; 1.3876x over previous
import functools

import jax
import jax.numpy as jnp
from jax.experimental import pallas as pl
from jax.experimental.pallas import tpu as pltpu

_B, _D, _E = 8192, 1024, 16
_TOPK = 2
_LOSS_COEF = 0.01


def _fused_kernel(x_ref, w1t_ref, b1_ref, wout_ref, bout_ref, coefs_ref,
                  icpt_ref, out_ref, loss_ref, gates_ref, imp_ref, load_ref,
                  *, n_steps):
    i = pl.program_id(0)

    x = x_ref[...]
    g = jnp.dot(x.astype(jnp.bfloat16), w1t_ref[...].astype(jnp.bfloat16),
                preferred_element_type=jnp.float32)
    g = jnp.maximum(g + b1_ref[...], 0.0)

    scores_t = jax.lax.dot_general(
        wout_ref[...], g, (((1,), (1,)), ((), ())),
        preferred_element_type=jnp.float32)
    scores_t = scores_t + bout_ref[...]

    e_idx = jax.lax.broadcasted_iota(jnp.int32, scores_t.shape, 0)
    m1 = jnp.max(scores_t, axis=0, keepdims=True)
    idx1 = jnp.min(jnp.where(scores_t == m1, e_idx, _E), axis=0, keepdims=True)
    sel1 = e_idx == idx1
    rest = jnp.where(sel1, -jnp.inf, scores_t)
    m2 = jnp.max(rest, axis=0, keepdims=True)
    idx2 = jnp.min(jnp.where(rest == m2, e_idx, _E), axis=0, keepdims=True)
    mask = sel1 | (e_idx == idx2)

    mx = jnp.maximum(m1, 0.0)
    em0 = jnp.exp(-mx)
    denom = jnp.exp(m1 - mx) + jnp.exp(m2 - mx) + (_E - _TOPK) * em0
    gates_t = jnp.where(mask, jnp.exp(scores_t - mx), em0) / denom

    gates_ref[...] = gates_t.T

    eo_t = jax.lax.dot_general(
        coefs_ref[...], x, (((1,), (1,)), ((), ())),
        preferred_element_type=jnp.float32)
    eo_t = eo_t + icpt_ref[...]
    out_ref[...] = jnp.sum(gates_t * eo_t, axis=0, keepdims=True)

    @pl.when(i == 0)
    def _init():
        imp_ref[...] = jnp.zeros_like(imp_ref)
        load_ref[...] = jnp.zeros_like(load_ref)

    imp_ref[...] += jnp.sum(gates_t, axis=1, keepdims=True)
    load_ref[...] += jnp.sum((gates_t > 0.0).astype(jnp.float32), axis=1,
                             keepdims=True)

    @pl.when(i == n_steps - 1)
    def _finish():
        def cv2(v):
            mean = jnp.sum(v) / _E
            var = jnp.sum((v - mean) ** 2) / (_E - 1)
            return var / (mean * mean + 1e-10)

        loss = (cv2(imp_ref[...]) + cv2(load_ref[...])) * _LOSS_COEF
        loss_ref[...] = loss.reshape(1, 1)


@jax.jit
def kernel(x, W1, b1, Wout, bout, coefs, intercepts):
    BT = 512
    n_steps = _B // BT

    w1t = W1.T
    b1r = b1.reshape(1, _D)
    boutr = bout.reshape(_E, 1)
    icptr = intercepts.reshape(_E, 1)

    out2d, loss2d, gates = pl.pallas_call(
        functools.partial(_fused_kernel, n_steps=n_steps),
        grid=(n_steps,),
        in_specs=[
            pl.BlockSpec((BT, _D), lambda i: (i, 0)),
            pl.BlockSpec((_D, _D), lambda i: (0, 0)),
            pl.BlockSpec((1, _D), lambda i: (0, 0)),
            pl.BlockSpec((_E, _D), lambda i: (0, 0)),
            pl.BlockSpec((_E, 1), lambda i: (0, 0)),
            pl.BlockSpec((_E, _D), lambda i: (0, 0)),
            pl.BlockSpec((_E, 1), lambda i: (0, 0)),
        ],
        out_specs=[
            pl.BlockSpec((1, BT), lambda i: (0, i)),
            pl.BlockSpec((1, 1), lambda i: (0, 0)),
            pl.BlockSpec((BT, _E), lambda i: (i, 0)),
        ],
        out_shape=[
            jax.ShapeDtypeStruct((1, _B), jnp.float32),
            jax.ShapeDtypeStruct((1, 1), jnp.float32),
            jax.ShapeDtypeStruct((_B, _E), jnp.float32),
        ],
        scratch_shapes=[
            pltpu.VMEM((_E, 1), jnp.float32),
            pltpu.VMEM((_E, 1), jnp.float32),
        ],
        compiler_params=pltpu.CompilerParams(
            dimension_semantics=("arbitrary",),
        ),
    )(x, w1t, b1r, Wout, boutr, coefs, icptr)

    return out2d.reshape(_B), loss2d[0, 0], gates
